# baseline (speedup 1.0000x reference)
import functools

import jax
import jax.numpy as jnp
from jax import lax
from jax.experimental import pallas as pl
from jax.experimental.pallas import tpu as pltpu
from jax.experimental.pallas import tpu_sc as plsc

_N, _C, _H, _W = 32, 68, 128, 128
_TOPK = 9
_STRIDE = 4.0
_HW = _H * _W
_ROWS = _N * _C
_L = 16
_CHUNK = 16
_NCHUNK = _HW // (_L * _CHUNK)
_NC, _NS = 2, 16
_NW = _NC * _NS
_RPW = _ROWS // _NW

_NEG = float("-inf")
_BIGI = 2**31 - 1


def _merge_topk(tv, ti, v, vi):
    sv, si = plsc.sort_key_val(v, vi, descending=True)
    take = (sv > tv) | ((sv == tv) & (si < ti))
    hv = jnp.where(take, sv, tv)
    hi = jnp.where(take, si, ti)
    rv, ri = plsc.sort_key_val(hv, hi)
    return rv, ri


def _row_topk(row_ref, taup):
    iota = lax.iota(jnp.int32, _L)

    def step(c, carry):
        tv, ti, tau = carry
        base = c * (_L * _CHUNK)
        vs = [row_ref[pl.ds(base + k * _L, _L)] for k in range(_CHUNK)]
        levels = [list(vs)]
        while len(levels[-1]) > 1:
            prev = levels[-1]
            levels.append([jnp.maximum(prev[2 * i], prev[2 * i + 1])
                           for i in range(len(prev) // 2)])
        mx = levels[-1][0]

        def on_hit(tv, ti, tau):
            acc = jnp.zeros((_L,), jnp.int32)
            for k in range(_CHUNK):
                acc = acc | jnp.where(vs[k] > tau, jnp.int32(1 << k), 0)
            for c in (1, 2, 4, 8):
                acc = acc | acc.at[jnp.bitwise_xor(iota, c)].get(
                    mode="promise_in_bounds")
            bits = jnp.max(acc)
            for k in range(_CHUNK):
                tv, ti = lax.cond(
                    jnp.bitwise_and(bits, 1 << k) != 0,
                    lambda tv, ti, k=k: _merge_topk(
                        tv, ti, vs[k], base + k * _L + iota),
                    lambda tv, ti: (tv, ti),
                    tv, ti)
            tau = jnp.maximum(taup, jnp.broadcast_to(jnp.min(tv), (_L,)))
            return tv, ti, tau

        return lax.cond(jnp.any(mx > tau), on_hit,
                        lambda tv, ti, tau: (tv, ti, tau), tv, ti, tau)

    tv0 = jnp.full((_L,), _NEG, jnp.float32)
    ti0 = jnp.full((_L,), _BIGI, jnp.int32)
    tv, ti, _ = lax.fori_loop(0, _NCHUNK, step, (tv0, ti0, taup))
    return tv, ti


def _finalize(tv, ti):
    sel = jnp.zeros((_L,), jnp.bool_)
    v = tv
    for _ in range(_TOPK):
        m = jnp.max(v)
        tie = v == m
        imin = jnp.min(jnp.where(tie, ti, jnp.full((_L,), _BIGI, jnp.int32)))
        pick = tie & (ti == imin)
        sel = sel | pick
        v = jnp.where(pick, _NEG, v)
    e = jnp.where(sel, jnp.exp(tv - jnp.max(tv)), jnp.float32(0.0))
    p = (e * _STRIDE) / jnp.broadcast_to(jnp.sum(e), (_L,))
    xf = jnp.bitwise_and(ti, _W - 1).astype(jnp.float32)
    yf = jnp.right_shift(ti, 7).astype(jnp.float32)
    return jnp.sum(p * xf), jnp.sum(p * yf)


@functools.partial(
    pl.kernel,
    out_type=jax.ShapeDtypeStruct((_ROWS * _L,), jnp.float32),
    mesh=plsc.VectorSubcoreMesh(core_axis_name="c", subcore_axis_name="s",
                                num_cores=_NC, num_subcores=_NS),
    scratch_types=[
        pltpu.VMEM((_HW,), jnp.float32),
        pltpu.VMEM((_HW,), jnp.float32),
        pltpu.VMEM((_RPW * _L,), jnp.float32),
        pltpu.SemaphoreType.DMA,
        pltpu.SemaphoreType.DMA,
    ],
    compiler_params=pltpu.CompilerParams(needs_layout_passes=False),
)
def _sc_topk_coord(x_hbm, out_hbm, buf0, buf1, outv, sem0, sem1):
    wid = lax.axis_index("s") * _NC + lax.axis_index("c")
    row0 = wid * _RPW
    bufs = (buf0, buf1)
    sems = (sem0, sem1)
    iota = lax.iota(jnp.int32, _L)
    for b in range(2):
        pltpu.make_async_copy(
            x_hbm.at[pl.ds((row0 + b) * _HW, _HW)], bufs[b], sems[b]).start()

    neg_vec = jnp.full((_L,), _NEG, jnp.float32)

    def outer(g, taup):
        for b in range(2):
            r = g * 2 + b
            pltpu.make_async_copy(
                x_hbm.at[pl.ds(0, _HW)], bufs[b], sems[b]).wait()
            tv, ti = _row_topk(bufs[b], taup)
            nreal = jnp.sum((tv > taup).astype(jnp.int32))
            tv, ti = lax.cond(
                nreal >= _TOPK,
                lambda tv, ti: (tv, ti),
                lambda tv, ti, ref=bufs[b]: _row_topk(ref, neg_vec),
                tv, ti)
            taup = jnp.broadcast_to(jnp.min(tv), (_L,))

            @pl.when(r + 2 < _RPW)
            def _():
                pltpu.make_async_copy(
                    x_hbm.at[pl.ds((row0 + r + 2) * _HW, _HW)],
                    bufs[b], sems[b]).start()

            ox, oy = _finalize(tv, ti)
            res = jnp.where(iota == 0, ox, jnp.where(iota == 1, oy, 0.0))
            outv[pl.ds(r * _L, _L)] = res
        return taup

    lax.fori_loop(0, _RPW // 2, outer, neg_vec)
    pltpu.sync_copy(outv, out_hbm.at[pl.ds(row0 * _L, _RPW * _L)])


def kernel(input):
    out = _sc_topk_coord(input.reshape(-1))
    return out.reshape(_ROWS, _L)[:, :2].reshape(_N, _C, 2)


# speedup vs baseline: 1.6981x; 1.6981x over previous
import functools

import jax
import jax.numpy as jnp
from jax import lax
from jax.experimental import pallas as pl
from jax.experimental.pallas import tpu as pltpu
from jax.experimental.pallas import tpu_sc as plsc

_N, _C, _H, _W = 32, 68, 128, 128
_TOPK = 9
_STRIDE = 4.0
_HW = _H * _W
_ROWS = _N * _C
_L = 16
_CHUNK = 16
_NCHUNK = _HW // (_L * _CHUNK)
_NC, _NS = 2, 16
_NW = _NC * _NS
_RPW = _ROWS // _NW

_NEG = float("-inf")
_BIGI = 2**31 - 1


def _merge_topk(tv, ti, v, vi):
    sv, si = plsc.sort_key_val(v, vi, descending=True)
    take = (sv > tv) | ((sv == tv) & (si < ti))
    hv = jnp.where(take, sv, tv)
    hi = jnp.where(take, si, ti)
    rv, ri = plsc.sort_key_val(hv, hi)
    return rv, ri


def _row_topk(row_ref, taup):
    iota = lax.iota(jnp.int32, _L)

    def step(c, carry):
        tv, ti, tau = carry
        base = c * (_L * _CHUNK)
        vs = [row_ref[pl.ds(base + k * _L, _L)] for k in range(_CHUNK)]
        levels = [list(vs)]
        while len(levels[-1]) > 1:
            prev = levels[-1]
            levels.append([jnp.maximum(prev[2 * i], prev[2 * i + 1])
                           for i in range(len(prev) // 2)])
        mx = levels[-1][0]

        def on_hit(tv, ti, tau):
            idxs = [base + k * _L + iota for k in range(_CHUNK)]
            v1, i1, s2 = [], [], []
            for i in range(_CHUNK // 2):
                a, bb = vs[2 * i], vs[2 * i + 1]
                ge = a >= bb
                v1.append(jnp.where(ge, a, bb))
                i1.append(jnp.where(ge, idxs[2 * i], idxs[2 * i + 1]))
                s2.append(jnp.minimum(a, bb))
            while len(v1) > 1:
                nv, ni, ns = [], [], []
                for i in range(len(v1) // 2):
                    a, bb = v1[2 * i], v1[2 * i + 1]
                    ge = a >= bb
                    nv.append(jnp.where(ge, a, bb))
                    ni.append(jnp.where(ge, i1[2 * i], i1[2 * i + 1]))
                    ns.append(jnp.maximum(
                        jnp.minimum(a, bb),
                        jnp.maximum(s2[2 * i], s2[2 * i + 1])))
                v1, i1, s2 = nv, ni, ns
            tv, ti = _merge_topk(tv, ti, v1[0], i1[0])

            def full(tv, ti):
                for k in range(_CHUNK):
                    tv, ti = lax.cond(
                        jnp.any(vs[k] > tau),
                        lambda tv, ti, k=k: _merge_topk(
                            tv, ti, vs[k], idxs[k]),
                        lambda tv, ti: (tv, ti),
                        tv, ti)
                return tv, ti

            tv, ti = lax.cond(jnp.any(s2[0] > tau), full,
                              lambda tv, ti: (tv, ti), tv, ti)
            tau = jnp.maximum(taup, jnp.broadcast_to(jnp.min(tv), (_L,)))
            return tv, ti, tau

        return lax.cond(jnp.any(mx > tau), on_hit,
                        lambda tv, ti, tau: (tv, ti, tau), tv, ti, tau)

    tv0 = jnp.full((_L,), _NEG, jnp.float32)
    ti0 = jnp.full((_L,), _BIGI, jnp.int32)
    tv, ti, _ = lax.fori_loop(0, _NCHUNK, step, (tv0, ti0, taup))
    return tv, ti


def _finalize(tv, ti):
    sel = jnp.zeros((_L,), jnp.bool_)
    v = tv
    for _ in range(_TOPK):
        m = jnp.max(v)
        tie = v == m
        imin = jnp.min(jnp.where(tie, ti, jnp.full((_L,), _BIGI, jnp.int32)))
        pick = tie & (ti == imin)
        sel = sel | pick
        v = jnp.where(pick, _NEG, v)
    e = jnp.where(sel, jnp.exp(tv - jnp.max(tv)), jnp.float32(0.0))
    p = (e * _STRIDE) / jnp.broadcast_to(jnp.sum(e), (_L,))
    xf = jnp.bitwise_and(ti, _W - 1).astype(jnp.float32)
    yf = jnp.right_shift(ti, 7).astype(jnp.float32)
    return jnp.sum(p * xf), jnp.sum(p * yf)


@functools.partial(
    pl.kernel,
    out_type=jax.ShapeDtypeStruct((_ROWS * _L,), jnp.float32),
    mesh=plsc.VectorSubcoreMesh(core_axis_name="c", subcore_axis_name="s",
                                num_cores=_NC, num_subcores=_NS),
    scratch_types=[
        pltpu.VMEM((_HW,), jnp.float32),
        pltpu.VMEM((_HW,), jnp.float32),
        pltpu.VMEM((_RPW * _L,), jnp.float32),
        pltpu.SemaphoreType.DMA,
        pltpu.SemaphoreType.DMA,
    ],
    compiler_params=pltpu.CompilerParams(needs_layout_passes=False),
)
def _sc_topk_coord(x_hbm, out_hbm, buf0, buf1, outv, sem0, sem1):
    wid = lax.axis_index("s") * _NC + lax.axis_index("c")
    row0 = wid * _RPW
    bufs = (buf0, buf1)
    sems = (sem0, sem1)
    iota = lax.iota(jnp.int32, _L)
    for b in range(2):
        pltpu.make_async_copy(
            x_hbm.at[pl.ds((row0 + b) * _HW, _HW)], bufs[b], sems[b]).start()

    neg_vec = jnp.full((_L,), _NEG, jnp.float32)

    def outer(g, carry):
        ta, tb = carry
        for b in range(2):
            r = g * 2 + b
            taup = jnp.minimum(ta, tb)
            pltpu.make_async_copy(
                x_hbm.at[pl.ds(0, _HW)], bufs[b], sems[b]).wait()
            tv, ti = _row_topk(bufs[b], taup)
            nreal = jnp.sum((tv > taup).astype(jnp.int32))
            tv, ti = lax.cond(
                nreal >= _TOPK,
                lambda tv, ti: (tv, ti),
                lambda tv, ti, ref=bufs[b]: _row_topk(ref, neg_vec),
                tv, ti)
            ta, tb = jnp.maximum(taup, jnp.broadcast_to(jnp.min(tv), (_L,))), ta

            @pl.when(r + 2 < _RPW)
            def _():
                pltpu.make_async_copy(
                    x_hbm.at[pl.ds((row0 + r + 2) * _HW, _HW)],
                    bufs[b], sems[b]).start()

            ox, oy = _finalize(tv, ti)
            res = jnp.where(iota == 0, ox, jnp.where(iota == 1, oy, 0.0))
            outv[pl.ds(r * _L, _L)] = res
        return ta, tb

    lax.fori_loop(0, _RPW // 2, outer, (neg_vec, neg_vec))
    pltpu.sync_copy(outv, out_hbm.at[pl.ds(row0 * _L, _RPW * _L)])


def kernel(input):
    out = _sc_topk_coord(input.reshape(-1))
    return out.reshape(_ROWS, _L)[:, :2].reshape(_N, _C, 2)
